# baseline (speedup 1.0000x reference)
import jax
import jax.numpy as jnp
from jax.experimental import pallas as pl

DIM = 384
DIM_HEAD = 64
HEADS = 8
TOP_K = 256
EPS = 1e-5
KM_ITERS = 5
INNER = DIM_HEAD * HEADS
B = 4
SEQ = 1024
BH = B * HEADS
CHUNK = 4096


def _cln(x, g, b):
    mean = jnp.mean(x, axis=1, keepdims=True)
    var = jnp.var(x, axis=1, keepdims=True)
    return (x - mean) / jnp.sqrt(var + EPS) * g + b


def _l2n(t):
    n = jnp.sqrt(jnp.sum(t * t, axis=1, keepdims=True))
    return t / jnp.maximum(n, 1e-12)


def _dot(a, b, ca, cb, precision=None):
    return jax.lax.dot_general(
        a, b, (((ca,), (cb,)), ((), ())),
        preferred_element_type=jnp.float32, precision=precision)


def _proj_kernel(qs_ref, ctx_ref, wq_ref, wkv_ref, q_ref, kv_ref):
    q_ref[0] = _dot(qs_ref[0], wq_ref[...], 0, 1)
    kv_ref[0] = _dot(ctx_ref[0], wkv_ref[...], 0, 1)


def _assign_kernel(p_ref, c_ref, pp_ref, cc_ref, a_ref, cnt_ref):
    i = pl.program_id(0)
    dots = _dot(p_ref[...], c_ref[...], 1, 1)
    d = pp_ref[...] - 2.0 * dots + cc_ref[...]
    am = jnp.argmin(d, axis=1, keepdims=True).astype(jnp.int32)
    a_ref[...] = am
    oh = (am == jax.lax.broadcasted_iota(jnp.int32, (CHUNK, TOP_K), 1)
          ).astype(jnp.float32)
    part = _dot(jnp.ones((1, CHUNK), jnp.float32), oh, 1, 0)

    @pl.when(i == 0)
    def _():
        cnt_ref[...] = part

    @pl.when(i > 0)
    def _():
        cnt_ref[...] = cnt_ref[...] + part


def _kdist_kernel(k_ref, c_ref, pp_ref, cc_ref, kd_ref):
    dots = _dot(k_ref[...], c_ref[...], 1, 1)
    d = pp_ref[...] - 2.0 * dots + cc_ref[...]
    am = jnp.argmin(d, axis=1, keepdims=True).astype(jnp.int32)
    oh = (am == jax.lax.broadcasted_iota(jnp.int32, (CHUNK, TOP_K), 1)
          ).astype(jnp.float32)
    centers = _dot(oh, c_ref[...], 1, 0,
                   precision=jax.lax.Precision.HIGHEST)
    kd_ref[...] = jnp.sum(jnp.abs(centers - k_ref[...]), axis=1, keepdims=True)


def _attn_kernel(kdc_ref, kdr_ref, q_ref, k_ref, v_ref, o_ref):
    kdc = kdc_ref[0]
    kdr = kdr_ref[0]
    lane = jax.lax.broadcasted_iota(jnp.int32, (SEQ, SEQ), 1)
    sub = jax.lax.broadcasted_iota(jnp.int32, (SEQ, SEQ), 0)
    gt = kdr > kdc
    tie = (kdr == kdc) & (lane < sub)
    m01 = (gt | tie).astype(jnp.float32)
    rank = _dot(m01, jnp.ones((1, SEQ), jnp.float32), 1, 1)
    rank = rank.astype(jnp.int32)
    scat = (rank == jax.lax.broadcasted_iota(jnp.int32, (SEQ, TOP_K), 1)
            ).astype(jnp.float32)
    ksel = _dot(k_ref[0], scat, 1, 0,
                precision=jax.lax.Precision.HIGHEST)
    vsel = _dot(v_ref[0], scat, 1, 0,
                precision=jax.lax.Precision.HIGHEST)
    sim = _dot(q_ref[0], ksel, 0, 0)
    m = jnp.max(sim, axis=1, keepdims=True)
    e = jnp.exp(sim - m)
    p = e / jnp.sum(e, axis=1, keepdims=True)
    o_ref[0] = _dot(vsel, p, 1, 1)


def _outproj_kernel(x_ref, w_ref, qs_ref, g_ref, b_ref, gm_ref, o_ref):
    o = _dot(w_ref[...], x_ref[0], 1, 0)
    mean = jnp.mean(o, axis=0, keepdims=True)
    var = jnp.mean((o - mean) * (o - mean), axis=0, keepdims=True)
    o = (o - mean) / jnp.sqrt(var + EPS) * g_ref[...] + b_ref[...]
    o_ref[0] = gm_ref[0, 0] * o + qs_ref[0]


def _full(s):
    return pl.BlockSpec(s, lambda i: tuple(0 for _ in s))


def _bat(s):
    return pl.BlockSpec(s, lambda i: (i,) + tuple(0 for _ in s[1:]))


def kernel(query_source, context, g_ctx, b_ctx, g_qs, b_qs, g_on, b_on,
           W_q, W_kv, W_out, gamma):
    b, c, H, W = query_source.shape

    ctxn = _cln(context, g_ctx, b_ctx)
    qsn = _cln(query_source, g_qs, b_qs)

    q3, kv3 = pl.pallas_call(
        _proj_kernel,
        grid=(B,),
        in_specs=[_bat((1, DIM, SEQ)), _bat((1, DIM, SEQ)),
                  _full((INNER, DIM)), _full((2 * INNER, DIM))],
        out_specs=[_bat((1, SEQ, INNER)), _bat((1, SEQ, 2 * INNER))],
        out_shape=[jax.ShapeDtypeStruct((B, SEQ, INNER), jnp.float32),
                   jax.ShapeDtypeStruct((B, SEQ, 2 * INNER), jnp.float32)],
    )(qsn.reshape(B, DIM, SEQ), ctxn.reshape(B, DIM, SEQ), W_q, W_kv)

    q4 = q3.transpose(0, 2, 1).reshape(B * HEADS, DIM_HEAD, H, W)
    kv4 = kv3.transpose(0, 2, 1).reshape(B, 2 * INNER, H, W)
    k4 = kv4[:, :INNER].reshape(B * HEADS, DIM_HEAD, H, W)
    v4 = kv4[:, INNER:].reshape(B * HEADS, DIM_HEAD, H, W)
    qn4 = _l2n(q4)
    kn4 = _l2n(k4)
    q_cm = qn4.reshape(BH, DIM_HEAD, SEQ)
    k_cm = kn4.reshape(BH, DIM_HEAD, SEQ)
    v_cm = v4.reshape(BH, DIM_HEAD, SEQ)

    points = jnp.transpose(qn4, (0, 2, 3, 1)).reshape(-1, DIM_HEAD)
    keys = jnp.transpose(kn4, (0, 2, 3, 1)).reshape(-1, DIM_HEAD)
    pp = jnp.sum(points * points, axis=-1, keepdims=True)

    nchunk = points.shape[0] // CHUNK
    cent = points[:TOP_K]
    for _ in range(KM_ITERS):
        cc = jnp.sum(cent * cent, axis=-1)
        assign, counts = pl.pallas_call(
            _assign_kernel,
            grid=(nchunk,),
            in_specs=[_bat((CHUNK, DIM_HEAD)), _full((TOP_K, DIM_HEAD)),
                      _bat((CHUNK, 1)), _full((1, TOP_K))],
            out_specs=[_bat((CHUNK, 1)), _full((1, TOP_K))],
            out_shape=[jax.ShapeDtypeStruct((points.shape[0], 1), jnp.int32),
                       jax.ShapeDtypeStruct((1, TOP_K), jnp.float32)],
        )(points, cent, pp, cc[None, :])
        assign = assign.reshape(-1)
        counts = counts.reshape(-1)
        sums = jax.ops.segment_sum(points, assign, num_segments=TOP_K)
        new_c = sums / jnp.maximum(counts, 1.0)[:, None]
        cent = jnp.where((counts > 0)[:, None], new_c, cent)

    cc = jnp.sum(cent * cent, axis=-1)
    ppk = jnp.sum(keys * keys, axis=-1, keepdims=True)
    kd = pl.pallas_call(
        _kdist_kernel,
        grid=(nchunk,),
        in_specs=[_bat((CHUNK, DIM_HEAD)), _full((TOP_K, DIM_HEAD)),
                  _bat((CHUNK, 1)), _full((1, TOP_K))],
        out_specs=_bat((CHUNK, 1)),
        out_shape=jax.ShapeDtypeStruct((keys.shape[0], 1), jnp.float32),
    )(keys, cent, ppk, cc[None, :])

    kdc = kd.reshape(BH, SEQ, 1)
    kdr = kd.reshape(BH, 1, SEQ)

    out_bh = pl.pallas_call(
        _attn_kernel,
        grid=(BH,),
        in_specs=[_bat((1, SEQ, 1)), _bat((1, 1, SEQ)),
                  _bat((1, DIM_HEAD, SEQ)), _bat((1, DIM_HEAD, SEQ)),
                  _bat((1, DIM_HEAD, SEQ))],
        out_specs=_bat((1, DIM_HEAD, SEQ)),
        out_shape=jax.ShapeDtypeStruct((BH, DIM_HEAD, SEQ), jnp.float32),
    )(kdc, kdr, q_cm, k_cm, v_cm)

    cat = out_bh.reshape(B, INNER, SEQ)

    o3 = pl.pallas_call(
        _outproj_kernel,
        grid=(B,),
        in_specs=[_bat((1, INNER, SEQ)), _full((DIM, INNER)),
                  _bat((1, DIM, SEQ)), _full((DIM, 1)), _full((DIM, 1)),
                  _full((1, 1))],
        out_specs=_bat((1, DIM, SEQ)),
        out_shape=jax.ShapeDtypeStruct((B, DIM, SEQ), jnp.float32),
    )(cat, W_out, query_source.reshape(B, DIM, SEQ),
      g_on.reshape(DIM, 1), b_on.reshape(DIM, 1), gamma.reshape(1, 1))

    return o3.reshape(b, c, H, W)


# speedup vs baseline: 1.0064x; 1.0064x over previous
import jax
import jax.numpy as jnp
from jax.experimental import pallas as pl

DIM = 384
DIM_HEAD = 64
HEADS = 8
TOP_K = 256
EPS = 1e-5
KM_ITERS = 5
INNER = DIM_HEAD * HEADS
B = 4
SEQ = 1024
BH = B * HEADS
CHUNK = 4096


def _cln(x, g, b):
    mean = jnp.mean(x, axis=1, keepdims=True)
    var = jnp.var(x, axis=1, keepdims=True)
    return (x - mean) / jnp.sqrt(var + EPS) * g + b


def _l2n(t):
    n = jnp.sqrt(jnp.sum(t * t, axis=1, keepdims=True))
    return t / jnp.maximum(n, 1e-12)


def _dot(a, b, ca, cb, precision=None):
    return jax.lax.dot_general(
        a, b, (((ca,), (cb,)), ((), ())),
        preferred_element_type=jnp.float32, precision=precision)


def _qproj_kernel(qs_ref, wq_ref, q_ref):
    q_ref[0] = _dot(qs_ref[0], wq_ref[...], 0, 1)


def _kvproj_kernel(ctx_ref, wkv_ref, kv_ref):
    kv_ref[0] = _dot(ctx_ref[0], wkv_ref[...], 0, 1)


def _assign_kernel(p_ref, c_ref, pp_ref, cc_ref, a_ref, cnt_ref):
    i = pl.program_id(0)
    dots = _dot(p_ref[...], c_ref[...], 1, 1)
    d = pp_ref[...] - 2.0 * dots + cc_ref[...]
    am = jnp.argmin(d, axis=1, keepdims=True).astype(jnp.int32)
    a_ref[...] = am
    oh = (am == jax.lax.broadcasted_iota(jnp.int32, (CHUNK, TOP_K), 1)
          ).astype(jnp.float32)
    part = _dot(jnp.ones((1, CHUNK), jnp.float32), oh, 1, 0)

    @pl.when(i == 0)
    def _():
        cnt_ref[...] = part

    @pl.when(i > 0)
    def _():
        cnt_ref[...] = cnt_ref[...] + part


def _kdist_kernel(k_ref, c_ref, pp_ref, cc_ref, kd_ref):
    dots = _dot(k_ref[...], c_ref[...], 1, 1)
    d = pp_ref[...] - 2.0 * dots + cc_ref[...]
    am = jnp.argmin(d, axis=1, keepdims=True).astype(jnp.int32)
    oh = (am == jax.lax.broadcasted_iota(jnp.int32, (CHUNK, TOP_K), 1)
          ).astype(jnp.float32)
    centers = _dot(oh, c_ref[...], 1, 0,
                   precision=jax.lax.Precision.HIGHEST)
    kd_ref[...] = jnp.sum(jnp.abs(centers - k_ref[...]), axis=1, keepdims=True)


def _attn_kernel(kdc_ref, kdr_ref, q_ref, k_ref, v_ref, o_ref):
    kdc = kdc_ref[0]
    kdr = kdr_ref[0]
    ones_row = jnp.ones((1, SEQ), jnp.float32)
    gt = (kdr > kdc).astype(jnp.float32)
    eq = (kdr == kdc).astype(jnp.float32)
    rank_gt = _dot(gt, ones_row, 1, 1)
    eq_cnt = _dot(eq, ones_row, 1, 1)
    has_tie = jnp.max(eq_cnt) > 1.0

    def _with_ties(eq):
        lane = jax.lax.broadcasted_iota(jnp.int32, (SEQ, SEQ), 1)
        sub = jax.lax.broadcasted_iota(jnp.int32, (SEQ, SEQ), 0)
        tie = eq * (lane < sub).astype(jnp.float32)
        return _dot(tie, ones_row, 1, 1)

    corr = jax.lax.cond(has_tie, _with_ties,
                        lambda eq: jnp.zeros((SEQ, 1), jnp.float32), eq)
    rank = (rank_gt + corr).astype(jnp.int32)
    scat = (rank == jax.lax.broadcasted_iota(jnp.int32, (SEQ, TOP_K), 1)
            ).astype(jnp.float32)
    ksel = _dot(k_ref[0], scat, 1, 0,
                precision=jax.lax.Precision.HIGHEST)
    vsel = _dot(v_ref[0], scat, 1, 0,
                precision=jax.lax.Precision.HIGHEST)
    sim = _dot(q_ref[0], ksel, 0, 0)
    m = jnp.max(sim, axis=1, keepdims=True)
    e = jnp.exp(sim - m)
    p = e / jnp.sum(e, axis=1, keepdims=True)
    o_ref[0] = _dot(vsel, p, 1, 1)


def _outproj_kernel(x_ref, w_ref, qs_ref, g_ref, b_ref, gm_ref, o_ref):
    o = _dot(w_ref[...], x_ref[0], 1, 0)
    mean = jnp.mean(o, axis=0, keepdims=True)
    var = jnp.mean((o - mean) * (o - mean), axis=0, keepdims=True)
    o = (o - mean) / jnp.sqrt(var + EPS) * g_ref[...] + b_ref[...]
    o_ref[0] = gm_ref[0, 0] * o + qs_ref[0]


def _full(s):
    return pl.BlockSpec(s, lambda i: tuple(0 for _ in s))


def _bat(s):
    return pl.BlockSpec(s, lambda i: (i,) + tuple(0 for _ in s[1:]))


def kernel(query_source, context, g_ctx, b_ctx, g_qs, b_qs, g_on, b_on,
           W_q, W_kv, W_out, gamma):
    b, c, H, W = query_source.shape

    ctxn = _cln(context, g_ctx, b_ctx)
    qsn = _cln(query_source, g_qs, b_qs)

    q3 = pl.pallas_call(
        _qproj_kernel,
        grid=(B,),
        in_specs=[_bat((1, DIM, SEQ)), _full((INNER, DIM))],
        out_specs=_bat((1, SEQ, INNER)),
        out_shape=jax.ShapeDtypeStruct((B, SEQ, INNER), jnp.float32),
    )(qsn.reshape(B, DIM, SEQ), W_q)

    kv3 = pl.pallas_call(
        _kvproj_kernel,
        grid=(B,),
        in_specs=[_bat((1, DIM, SEQ)), _full((2 * INNER, DIM))],
        out_specs=_bat((1, SEQ, 2 * INNER)),
        out_shape=jax.ShapeDtypeStruct((B, SEQ, 2 * INNER), jnp.float32),
    )(ctxn.reshape(B, DIM, SEQ), W_kv)

    q4 = q3.transpose(0, 2, 1).reshape(B * HEADS, DIM_HEAD, H, W)
    kv4 = kv3.transpose(0, 2, 1).reshape(B, 2 * INNER, H, W)
    k4 = kv4[:, :INNER].reshape(B * HEADS, DIM_HEAD, H, W)
    v4 = kv4[:, INNER:].reshape(B * HEADS, DIM_HEAD, H, W)
    qn4 = _l2n(q4)
    kn4 = _l2n(k4)
    q_cm = qn4.reshape(BH, DIM_HEAD, SEQ)
    k_cm = kn4.reshape(BH, DIM_HEAD, SEQ)
    v_cm = v4.reshape(BH, DIM_HEAD, SEQ)

    points = jnp.transpose(qn4, (0, 2, 3, 1)).reshape(-1, DIM_HEAD)
    keys = jnp.transpose(kn4, (0, 2, 3, 1)).reshape(-1, DIM_HEAD)
    pp = jnp.sum(points * points, axis=-1, keepdims=True)

    nchunk = points.shape[0] // CHUNK
    cent = points[:TOP_K]
    for _ in range(KM_ITERS):
        cc = jnp.sum(cent * cent, axis=-1)
        assign, counts = pl.pallas_call(
            _assign_kernel,
            grid=(nchunk,),
            in_specs=[_bat((CHUNK, DIM_HEAD)), _full((TOP_K, DIM_HEAD)),
                      _bat((CHUNK, 1)), _full((1, TOP_K))],
            out_specs=[_bat((CHUNK, 1)), _full((1, TOP_K))],
            out_shape=[jax.ShapeDtypeStruct((points.shape[0], 1), jnp.int32),
                       jax.ShapeDtypeStruct((1, TOP_K), jnp.float32)],
        )(points, cent, pp, cc[None, :])
        assign = assign.reshape(-1)
        counts = counts.reshape(-1)
        sums = jax.ops.segment_sum(points, assign, num_segments=TOP_K)
        new_c = sums / jnp.maximum(counts, 1.0)[:, None]
        cent = jnp.where((counts > 0)[:, None], new_c, cent)

    cc = jnp.sum(cent * cent, axis=-1)
    ppk = jnp.sum(keys * keys, axis=-1, keepdims=True)
    kd = pl.pallas_call(
        _kdist_kernel,
        grid=(nchunk,),
        in_specs=[_bat((CHUNK, DIM_HEAD)), _full((TOP_K, DIM_HEAD)),
                  _bat((CHUNK, 1)), _full((1, TOP_K))],
        out_specs=_bat((CHUNK, 1)),
        out_shape=jax.ShapeDtypeStruct((keys.shape[0], 1), jnp.float32),
    )(keys, cent, ppk, cc[None, :])

    kdc = kd.reshape(BH, SEQ, 1)
    kdr = kd.reshape(BH, 1, SEQ)

    out_bh = pl.pallas_call(
        _attn_kernel,
        grid=(BH,),
        in_specs=[_bat((1, SEQ, 1)), _bat((1, 1, SEQ)),
                  _bat((1, DIM_HEAD, SEQ)), _bat((1, DIM_HEAD, SEQ)),
                  _bat((1, DIM_HEAD, SEQ))],
        out_specs=_bat((1, DIM_HEAD, SEQ)),
        out_shape=jax.ShapeDtypeStruct((BH, DIM_HEAD, SEQ), jnp.float32),
    )(kdc, kdr, q_cm, k_cm, v_cm)

    cat = out_bh.reshape(B, INNER, SEQ)

    o3 = pl.pallas_call(
        _outproj_kernel,
        grid=(B,),
        in_specs=[_bat((1, INNER, SEQ)), _full((DIM, INNER)),
                  _bat((1, DIM, SEQ)), _full((DIM, 1)), _full((DIM, 1)),
                  _full((1, 1))],
        out_specs=_bat((1, DIM, SEQ)),
        out_shape=jax.ShapeDtypeStruct((B, DIM, SEQ), jnp.float32),
    )(cat, W_out, query_source.reshape(B, DIM, SEQ),
      g_on.reshape(DIM, 1), b_on.reshape(DIM, 1), gamma.reshape(1, 1))

    return o3.reshape(b, c, H, W)
